# baseline (speedup 1.0000x reference)
import functools
import math

import jax
import jax.numpy as jnp
from jax import lax
from jax.experimental import pallas as pl
from jax.experimental.pallas import tpu as pltpu
from jax.experimental.pallas import tpu_sc as plsc

B = 4096
K = 32
H = 128
BB = 256

NC, NS = 2, 16
NW = NC * NS
CH = 128
def _sc_gather_body(nt_small, nch_big, user_emb, item_emb, review_emb,
                    ix_uthis, ix_ithis, ix_und, ix_uns, ix_url,
                    ix_ind, ix_ins, ix_irl,
                    o_uthis, o_ithis, o_und, o_uns, o_url, o_ind, o_ins, o_irl,
                    idx_v, rows0, rows1, rows2, rows3,
                    gs0, gs1, gs2, gs3, ws0, ws1, ws2, ws3):
    w = lax.axis_index("s") * NC + lax.axis_index("c")
    rows = (rows0, rows1, rows2, rows3)
    gs = (gs0, gs1, gs2, gs3)
    ws = (ws0, ws1, ws2, ws3)
    npair = nch_big // 2
    big_jobs = (
        (item_emb, ix_und, o_und),
        (user_emb, ix_uns, o_uns),
        (review_emb, ix_url, o_url),
        (user_emb, ix_ind, o_ind),
        (item_emb, ix_ins, o_ins),
        (review_emb, ix_irl, o_irl),
    )
    for table, ix_hbm, out_hbm in big_jobs:
        pltpu.sync_copy(ix_hbm.at[w], idx_v.at[pl.ds(0, nch_big)])
        base = w * (nch_big * CH)

        def gstart(p, b):
            return (pltpu.async_copy(table.at[idx_v.at[2 * p]], rows[2 * b], gs[2 * b]),
                    pltpu.async_copy(table.at[idx_v.at[2 * p + 1]], rows[2 * b + 1], gs[2 * b + 1]))

        ga = gstart(0, 0)
        gb = gstart(1, 1)

        def body(p2, _):
            for g in (0, 1):
                p = 2 * p2 + g
                x0, x1 = 2 * g, 2 * g + 1
                pltpu.make_async_copy(table.at[idx_v.at[0]], rows[x0], gs[x0]).wait()
                pltpu.make_async_copy(table.at[idx_v.at[0]], rows[x1], gs[x1]).wait()
                w0 = pltpu.async_copy(rows[x0], out_hbm.at[pl.ds(base + 2 * p * CH, CH)], ws[x0])
                w1 = pltpu.async_copy(rows[x1], out_hbm.at[pl.ds(base + (2 * p + 1) * CH, CH)], ws[x1])
                w0.wait()
                w1.wait()
                pn = jnp.where(p + 2 >= npair, p + 2 - npair, p + 2)
                gstart(pn, g)
            return ()
        lax.fori_loop(0, npair // 2, body, (), unroll=False)
        for x in range(4):
            pltpu.make_async_copy(table.at[idx_v.at[0]], rows[x], gs[x]).wait()

    wm = jnp.minimum(w, nt_small - 1)
    for table, ix_hbm, out_hbm in ((user_emb, ix_uthis, o_uthis),
                                   (item_emb, ix_ithis, o_ithis)):
        @pl.when(w < nt_small)
        def _():
            pltpu.sync_copy(ix_hbm.at[wm], idx_v.at[pl.ds(0, 1)])
            pltpu.async_copy(table.at[idx_v.at[0]], rows0, gs0).wait()
            pltpu.sync_copy(rows0, out_hbm.at[pl.ds(wm * CH, CH)])


def _sc_gather_all(user_emb, item_emb, review_emb,
                   users_ind, items_ind, u_ne_items, u_ne_users, u_review_ids,
                   i_ne_users, i_ne_items, i_review_ids):
    n = users_ind.shape[0]
    nt_small = n // CH
    nch_big = (n * K) // (NW * CH)
    mesh = plsc.VectorSubcoreMesh(core_axis_name="c", subcore_axis_name="s",
                                  num_cores=NC, num_subcores=NS)
    small = jax.ShapeDtypeStruct((n, H), jnp.float32)
    big = jax.ShapeDtypeStruct((n * K, H), jnp.float32)
    fn = pl.kernel(
        functools.partial(_sc_gather_body, nt_small, nch_big),
        out_type=(small, small, big, big, big, big, big, big),
        mesh=mesh,
        scratch_types=(
            [pltpu.VMEM((nch_big, CH), jnp.int32)]
            + [pltpu.VMEM((CH, H), jnp.float32)] * 4
            + [pltpu.SemaphoreType.DMA] * 8
        ),
    )
    return fn(user_emb, item_emb, review_emb,
              users_ind.reshape(nt_small, 1, CH), items_ind.reshape(nt_small, 1, CH),
              *(a.reshape(NW, nch_big, CH) for a in
                (u_ne_items, u_ne_users, u_review_ids,
                 i_ne_users, i_ne_items, i_review_ids)))


def _ln(x, g, b):
    m = jnp.mean(x, axis=-1, keepdims=True)
    v = jnp.mean((x - m) ** 2, axis=-1, keepdims=True)
    return (x - m) * jax.lax.rsqrt(v + 1e-5) * g + b


def _side(this, nd, ns, rel):
    inv = 1.0 / math.sqrt(H)
    sd = jnp.sum(this[:, None, :] * rel, axis=-1) * inv
    ss = jnp.sum((this * this)[:, None, :] * ns, axis=-1) * inv
    m = jnp.maximum(jnp.max(sd, axis=-1, keepdims=True),
                    jnp.max(ss, axis=-1, keepdims=True))
    ed = jnp.exp(sd - m)
    es = jnp.exp(ss - m)
    z = jnp.sum(ed, axis=-1, keepdims=True) + jnp.sum(es, axis=-1, keepdims=True)
    pref = (jnp.sum(ed[:, :, None] * nd, axis=1)
            + jnp.sum(es[:, :, None] * ns, axis=1)) / z
    return jnp.concatenate([this, pref], axis=-1)


def _transform(x, W1, b1, W2, b2, g1, be1, g2, be2):
    h1 = jnp.maximum(jnp.dot(x, W1, preferred_element_type=jnp.float32) + b1, 0.0)
    h1 = _ln(h1, g1, be1)
    h2 = jnp.maximum(jnp.dot(h1, W2, preferred_element_type=jnp.float32) + b2, 0.0)
    return _ln(h2, g2, be2)


def _gat_block(u_this_ref, i_this_ref, u_nd_ref, u_ns_ref, u_rel_ref,
               i_nd_ref, i_ns_ref, i_rel_ref,
               W1_ref, b1_ref, W2_ref, b2_ref, g1_ref, be1_ref, g2_ref, be2_ref,
               up_ref, ip_ref, rp_ref):
    u_vec = _side(u_this_ref[...], u_nd_ref[...], u_ns_ref[...], u_rel_ref[...])
    i_vec = _side(i_this_ref[...], i_nd_ref[...], i_ns_ref[...], i_rel_ref[...])
    args = (W1_ref[...], b1_ref[...], W2_ref[...], b2_ref[...],
            g1_ref[...], be1_ref[...], g2_ref[...], be2_ref[...])
    up = _transform(u_vec, *args)
    ip = _transform(i_vec, *args)
    up_ref[...] = up
    ip_ref[...] = ip
    rp_ref[...] = up * ip


def _gat_tc(u_this, i_this, u_nd, u_ns, u_rel, i_nd, i_ns, i_rel,
            W1, b1, W2, b2, g1, be1, g2, be2):
    n = u_this.shape[0]
    nblk = n // BB
    bspec2 = pl.BlockSpec((BB, H), lambda i: (i, 0))
    bspec3 = pl.BlockSpec((BB, K, H), lambda i: (i, 0, 0))
    wfull = lambda s: pl.BlockSpec(s, lambda i: tuple(0 for _ in s))
    out_shapes = [jax.ShapeDtypeStruct((n, H), jnp.float32)] * 3
    return pl.pallas_call(
        _gat_block,
        grid=(nblk,),
        in_specs=[bspec2, bspec2, bspec3, bspec3, bspec3, bspec3, bspec3, bspec3,
                  wfull((2 * H, H)), wfull((1, H)), wfull((H, H)), wfull((1, H)),
                  wfull((1, H)), wfull((1, H)), wfull((1, H)), wfull((1, H))],
        out_specs=[bspec2, bspec2, bspec2],
        out_shape=out_shapes,
    )(u_this, i_this, u_nd, u_ns, u_rel, i_nd, i_ns, i_rel,
      W1, b1.reshape(1, H), W2, b2.reshape(1, H),
      g1.reshape(1, H), be1.reshape(1, H), g2.reshape(1, H), be2.reshape(1, H))


NSPLIT = 4


def kernel(users_ind, items_ind, u_ne_items, u_ne_users, i_ne_users, i_ne_items,
           u_review_ids, i_review_ids, user_emb, item_emb, review_emb,
           W1, b1, W2, b2, g1, be1, g2, be2):
    bs = B // NSPLIT
    gathered = []
    for s in range(NSPLIT):
        sl = slice(s * bs, (s + 1) * bs)
        gathered.append(_sc_gather_all(
            user_emb, item_emb, review_emb,
            users_ind[sl], items_ind[sl], u_ne_items[sl], u_ne_users[sl],
            u_review_ids[sl], i_ne_users[sl], i_ne_items[sl], i_review_ids[sl]))
    outs = []
    for s in range(NSPLIT):
        (u_this, i_this, u_nd, u_ns, u_rel, i_nd, i_ns, i_rel) = gathered[s]
        r3 = lambda a: a.reshape(bs, K, H)
        outs.append(_gat_tc(u_this, i_this, r3(u_nd), r3(u_ns), r3(u_rel),
                            r3(i_nd), r3(i_ns), r3(i_rel),
                            W1, b1, W2, b2, g1, be1, g2, be2))
    return tuple(jnp.concatenate([o[j] for o in outs], axis=0) for j in range(3))


# speedup vs baseline: 1.1037x; 1.1037x over previous
import functools
import math

import jax
import jax.numpy as jnp
from jax import lax
from jax.experimental import pallas as pl
from jax.experimental.pallas import tpu as pltpu
from jax.experimental.pallas import tpu_sc as plsc

B = 4096
K = 32
H = 128
BB = 256

NC, NS = 2, 16
NW = NC * NS
CH = 128
CR = 64
L = 16
NSPLIT = 2


def _dot_scores(this_v, rows_v, scores_v, b0, nb, lane0):
    iota = lax.iota(jnp.int32, L)
    shuf = [iota ^ sh for sh in (8, 4, 2, 1)]
    lane_eq = [iota == j for j in range(L)]
    for bl in range(nb):
        b = b0 + bl
        t = [this_v[b, pl.ds(L * c, L)] for c in range(H // L)]
        for kg in range(K // L):
            svec = jnp.zeros((L,), jnp.float32)
            for j in range(L):
                r = bl * K + kg * L + j
                acc = t[0] * rows_v[r, pl.ds(0, L)]
                for c in range(1, H // L):
                    acc += t[c] * rows_v[r, pl.ds(L * c, L)]
                for sv in shuf:
                    acc = acc + lax.gather(
                        acc, sv[:, None],
                        lax.GatherDimensionNumbers(offset_dims=(),
                                                   collapsed_slice_dims=(0,),
                                                   start_index_map=(0,)),
                        slice_sizes=(1,),
                        mode=lax.GatherScatterMode.PROMISE_IN_BOUNDS)
                svec = jnp.where(lane_eq[j], acc, svec)
            scores_v[pl.ds(b * K + kg * L, L)] = svec


def _sc_gather_body(m, nch_ne, nch_rel, user_emb, item_emb, review_emb,
                    ix_uthis, ix_ithis, ix_und, ix_uns, ix_url,
                    ix_ind, ix_ins, ix_irl,
                    o_uthis, o_ithis, o_und, o_uns, o_ind, o_ins, o_usd, o_isd,
                    idx_v, idxr_v, idxs_v, uthis_v, ithis_v, scores_v,
                    rows0, rows1, rows2, rows3,
                    gs0, gs1, gs2, gs3, ws0, ws1, ws2, ws3):
    w = lax.axis_index("s") * NC + lax.axis_index("c")
    rows = (rows0, rows1, rows2, rows3)
    gs = (gs0, gs1, gs2, gs3)
    ws = (ws0, ws1, ws2, ws3)
    lane0 = lax.iota(jnp.int32, L) == 0

    for table, ixs, out, this_v, sem in (
            (user_emb, ix_uthis, o_uthis, uthis_v, gs0),
            (item_emb, ix_ithis, o_ithis, ithis_v, gs1)):
        pltpu.sync_copy(ixs.at[w], idxs_v)
        pltpu.async_copy(table.at[idxs_v.at[0]], this_v, sem).wait()
        pltpu.sync_copy(this_v, out.at[pl.ds(w * m, m)])

    nbr = CR // K
    for ix_hbm, osd, this_v in ((ix_url, o_usd, uthis_v),
                                (ix_irl, o_isd, ithis_v)):
        pltpu.sync_copy(ix_hbm.at[w], idxr_v.at[pl.ds(0, nch_rel)])

        def rstart(ch, b):
            return pltpu.async_copy(review_emb.at[idxr_v.at[ch]],
                                    rows[b].at[pl.ds(0, CR)], gs[b])

        rstart(0, 0)

        def rbody(c2, _):
            for g in (0, 1):
                ch = 2 * c2 + g
                chn = jnp.where(ch + 1 >= nch_rel, 0, ch + 1)
                rstart(chn, 1 - g)
                pltpu.make_async_copy(review_emb.at[idxr_v.at[0]],
                                      rows[g].at[pl.ds(0, CR)], gs[g]).wait()
                _dot_scores(this_v, rows[g], scores_v, ch * nbr, nbr, lane0)
            return ()
        lax.fori_loop(0, nch_rel // 2, rbody, (), unroll=False)
        pltpu.make_async_copy(review_emb.at[idxr_v.at[0]],
                              rows[0].at[pl.ds(0, CR)], gs[0]).wait()
        pltpu.sync_copy(scores_v, osd.at[pl.ds(w * m * K, m * K)])

    npair = nch_ne // 2
    for table, ix_hbm, out_hbm in ((item_emb, ix_und, o_und),
                                   (user_emb, ix_uns, o_uns),
                                   (user_emb, ix_ind, o_ind),
                                   (item_emb, ix_ins, o_ins)):
        pltpu.sync_copy(ix_hbm.at[w], idx_v.at[pl.ds(0, nch_ne)])
        base = w * (nch_ne * CH)

        def gstart(p, b):
            return (pltpu.async_copy(table.at[idx_v.at[2 * p]], rows[2 * b], gs[2 * b]),
                    pltpu.async_copy(table.at[idx_v.at[2 * p + 1]], rows[2 * b + 1], gs[2 * b + 1]))

        gstart(0, 0)
        gstart(1, 1)

        def body(p2, _):
            for g in (0, 1):
                p = 2 * p2 + g
                x0, x1 = 2 * g, 2 * g + 1
                pltpu.make_async_copy(table.at[idx_v.at[0]], rows[x0], gs[x0]).wait()
                pltpu.make_async_copy(table.at[idx_v.at[0]], rows[x1], gs[x1]).wait()
                w0 = pltpu.async_copy(rows[x0], out_hbm.at[pl.ds(base + 2 * p * CH, CH)], ws[x0])
                w1 = pltpu.async_copy(rows[x1], out_hbm.at[pl.ds(base + (2 * p + 1) * CH, CH)], ws[x1])
                w0.wait()
                w1.wait()
                pn = jnp.where(p + 2 >= npair, p + 2 - npair, p + 2)
                gstart(pn, g)
            return ()
        lax.fori_loop(0, npair // 2, body, (), unroll=False)
        for x in range(4):
            pltpu.make_async_copy(table.at[idx_v.at[0]], rows[x], gs[x]).wait()


def _sc_gather_all(user_emb, item_emb, review_emb,
                   users_ind, items_ind, u_ne_items, u_ne_users, u_review_ids,
                   i_ne_users, i_ne_items, i_review_ids):
    n = users_ind.shape[0]
    m = n // NW
    nch_ne = (n * K) // (NW * CH)
    nch_rel = (n * K) // (NW * CR)
    mesh = plsc.VectorSubcoreMesh(core_axis_name="c", subcore_axis_name="s",
                                  num_cores=NC, num_subcores=NS)
    small = jax.ShapeDtypeStruct((n, H), jnp.float32)
    big = jax.ShapeDtypeStruct((n * K, H), jnp.float32)
    sd = jax.ShapeDtypeStruct((n * K,), jnp.float32)
    fn = pl.kernel(
        functools.partial(_sc_gather_body, m, nch_ne, nch_rel),
        out_type=(small, small, big, big, big, big, sd, sd),
        mesh=mesh,
        scratch_types=(
            [pltpu.VMEM((nch_ne, CH), jnp.int32),
             pltpu.VMEM((nch_rel, CR), jnp.int32),
             pltpu.VMEM((1, m), jnp.int32),
             pltpu.VMEM((m, H), jnp.float32),
             pltpu.VMEM((m, H), jnp.float32),
             pltpu.VMEM((m * K,), jnp.float32)]
            + [pltpu.VMEM((CH, H), jnp.float32)] * 4
            + [pltpu.SemaphoreType.DMA] * 8
        ),
    )
    return fn(user_emb, item_emb, review_emb,
              users_ind.reshape(NW, 1, m), items_ind.reshape(NW, 1, m),
              u_ne_items.reshape(NW, nch_ne, CH),
              u_ne_users.reshape(NW, nch_ne, CH),
              u_review_ids.reshape(NW, nch_rel, CR),
              i_ne_users.reshape(NW, nch_ne, CH),
              i_ne_items.reshape(NW, nch_ne, CH),
              i_review_ids.reshape(NW, nch_rel, CR))


def _ln(x, g, b):
    m = jnp.mean(x, axis=-1, keepdims=True)
    v = jnp.mean((x - m) ** 2, axis=-1, keepdims=True)
    return (x - m) * jax.lax.rsqrt(v + 1e-5) * g + b


def _side(this, nd, ns, sd_raw):
    inv = 1.0 / math.sqrt(H)
    sd = sd_raw * inv
    ss = jnp.sum((this * this)[:, None, :] * ns, axis=-1) * inv
    m = jnp.maximum(jnp.max(sd, axis=-1, keepdims=True),
                    jnp.max(ss, axis=-1, keepdims=True))
    ed = jnp.exp(sd - m)
    es = jnp.exp(ss - m)
    z = jnp.sum(ed, axis=-1, keepdims=True) + jnp.sum(es, axis=-1, keepdims=True)
    pref = (jnp.sum(ed[:, :, None] * nd, axis=1)
            + jnp.sum(es[:, :, None] * ns, axis=1)) / z
    return jnp.concatenate([this, pref], axis=-1)


def _transform(x, W1, b1, W2, b2, g1, be1, g2, be2):
    h1 = jnp.maximum(jnp.dot(x, W1, preferred_element_type=jnp.float32) + b1, 0.0)
    h1 = _ln(h1, g1, be1)
    h2 = jnp.maximum(jnp.dot(h1, W2, preferred_element_type=jnp.float32) + b2, 0.0)
    return _ln(h2, g2, be2)


def _gat_block(u_this_ref, i_this_ref, u_nd_ref, u_ns_ref, i_nd_ref, i_ns_ref,
               u_sd_ref, i_sd_ref,
               W1_ref, b1_ref, W2_ref, b2_ref, g1_ref, be1_ref, g2_ref, be2_ref,
               up_ref, ip_ref, rp_ref):
    u_vec = _side(u_this_ref[...], u_nd_ref[...], u_ns_ref[...], u_sd_ref[...])
    i_vec = _side(i_this_ref[...], i_nd_ref[...], i_ns_ref[...], i_sd_ref[...])
    args = (W1_ref[...], b1_ref[...], W2_ref[...], b2_ref[...],
            g1_ref[...], be1_ref[...], g2_ref[...], be2_ref[...])
    up = _transform(u_vec, *args)
    ip = _transform(i_vec, *args)
    up_ref[...] = up
    ip_ref[...] = ip
    rp_ref[...] = up * ip


def _gat_tc(u_this, i_this, u_nd, u_ns, i_nd, i_ns, u_sd, i_sd,
            W1, b1, W2, b2, g1, be1, g2, be2):
    n = u_this.shape[0]
    nblk = n // BB
    bspec2 = pl.BlockSpec((BB, H), lambda i: (i, 0))
    bspec3 = pl.BlockSpec((BB, K, H), lambda i: (i, 0, 0))
    bspecs = pl.BlockSpec((BB, K), lambda i: (i, 0))
    wfull = lambda s: pl.BlockSpec(s, lambda i: tuple(0 for _ in s))
    out_shapes = [jax.ShapeDtypeStruct((n, H), jnp.float32)] * 3
    return pl.pallas_call(
        _gat_block,
        grid=(nblk,),
        in_specs=[bspec2, bspec2, bspec3, bspec3, bspec3, bspec3,
                  bspecs, bspecs,
                  wfull((2 * H, H)), wfull((1, H)), wfull((H, H)), wfull((1, H)),
                  wfull((1, H)), wfull((1, H)), wfull((1, H)), wfull((1, H))],
        out_specs=[bspec2, bspec2, bspec2],
        out_shape=out_shapes,
    )(u_this, i_this, u_nd, u_ns, i_nd, i_ns, u_sd, i_sd,
      W1, b1.reshape(1, H), W2, b2.reshape(1, H),
      g1.reshape(1, H), be1.reshape(1, H), g2.reshape(1, H), be2.reshape(1, H))


def kernel(users_ind, items_ind, u_ne_items, u_ne_users, i_ne_users, i_ne_items,
           u_review_ids, i_review_ids, user_emb, item_emb, review_emb,
           W1, b1, W2, b2, g1, be1, g2, be2):
    bs = B // NSPLIT
    gathered = []
    for s in range(NSPLIT):
        sl = slice(s * bs, (s + 1) * bs)
        gathered.append(_sc_gather_all(
            user_emb, item_emb, review_emb,
            users_ind[sl], items_ind[sl], u_ne_items[sl], u_ne_users[sl],
            u_review_ids[sl], i_ne_users[sl], i_ne_items[sl], i_review_ids[sl]))
    outs = []
    for s in range(NSPLIT):
        (u_this, i_this, u_nd, u_ns, i_nd, i_ns, u_sd, i_sd) = gathered[s]
        r3 = lambda a: a.reshape(bs, K, H)
        outs.append(_gat_tc(u_this, i_this, r3(u_nd), r3(u_ns),
                            r3(i_nd), r3(i_ns),
                            u_sd.reshape(bs, K), i_sd.reshape(bs, K),
                            W1, b1, W2, b2, g1, be1, g2, be2))
    return tuple(jnp.concatenate([o[j] for o in outs], axis=0) for j in range(3))


# speedup vs baseline: 1.2126x; 1.0987x over previous
import functools
import math

import jax
import jax.numpy as jnp
from jax import lax
from jax.experimental import pallas as pl
from jax.experimental.pallas import tpu as pltpu
from jax.experimental.pallas import tpu_sc as plsc

B = 4096
K = 32
H = 128
BB = 256

NC, NS = 2, 16
NW = NC * NS
CH = 128
CR = 64
L = 16
NSPLIT = 2


def _dot_scores(this_v, rows_v, scores_v, tpose_v, b0, nb):
    iota = lax.iota(jnp.int32, L)
    dn = lax.GatherDimensionNumbers(offset_dims=(), collapsed_slice_dims=(0,),
                                    start_index_map=(0,))

    def _shuf(x, sv):
        return lax.gather(x, sv[:, None], dn, slice_sizes=(1,),
                          mode=lax.GatherScatterMode.PROMISE_IN_BOUNDS)

    levels = [((iota & sh) == 0, iota ^ sh) for sh in (1, 2, 4, 8)]
    nkg = K // L

    def kgroup(u, _):
        bl = u >> 1 if nkg == 2 else u // nkg
        kg = u & 1 if nkg == 2 else u % nkg
        b = b0 + bl
        t = [this_v[b, pl.ds(L * c, L)] for c in range(H // L)]
        r0 = bl * K + kg * L
        vs = []
        for j in range(L):
            acc = t[0] * rows_v[r0 + j, pl.ds(0, L)]
            for c in range(1, H // L):
                acc += t[c] * rows_v[r0 + j, pl.ds(L * c, L)]
            vs.append(acc)
        for mlane, sv in levels:
            vs = [jnp.where(mlane, vs[i], vs[i + 1])
                  + _shuf(jnp.where(mlane, vs[i + 1], vs[i]), sv)
                  for i in range(0, len(vs), 2)]
        scores_v[pl.ds(b * K + kg * L, L)] = vs[0]
        return ()

    lax.fori_loop(0, nb * nkg, kgroup, (), unroll=False)


def _sc_gather_body(m, nch_ne, nch_rel, user_emb, item_emb, review_emb,
                    ix_uthis, ix_ithis, ix_und, ix_uns, ix_url,
                    ix_ind, ix_ins, ix_irl,
                    o_uthis, o_ithis, o_und, o_uns, o_ind, o_ins, o_usd, o_isd,
                    idx_v, idxr_v, idxs_v, uthis_v, ithis_v, scores_v, tpose_v,
                    rows0, rows1, rows2, rows3, rel0, rel1,
                    gs0, gs1, gs2, gs3, ws0, ws1, ws2, ws3, rs0, rs1):
    w = lax.axis_index("s") * NC + lax.axis_index("c")
    rows = (rows0, rows1, rows2, rows3)
    gs = (gs0, gs1, gs2, gs3)
    ws = (ws0, ws1, ws2, ws3)

    for table, ixs, out, this_v, sem in (
            (user_emb, ix_uthis, o_uthis, uthis_v, gs0),
            (item_emb, ix_ithis, o_ithis, ithis_v, gs1)):
        pltpu.sync_copy(ixs.at[w], idxs_v)
        pltpu.async_copy(table.at[idxs_v.at[0]], this_v, sem).wait()
        pltpu.sync_copy(this_v, out.at[pl.ds(w * m, m)])

    pltpu.sync_copy(ix_url.at[w], idxr_v.at[pl.ds(0, nch_rel)])
    pltpu.sync_copy(ix_irl.at[w], idxr_v.at[pl.ds(nch_rel, nch_rel)])
    nbr = CR // K
    npair = nch_ne // 2
    RC = nch_rel
    rel = (rel0, rel1)
    rsem = (rs0, rs1)

    def rstart(row, q):
        return pltpu.async_copy(review_emb.at[idxr_v.at[row]], rel[q], rsem[q])

    def rwait(q):
        pltpu.make_async_copy(review_emb.at[idxr_v.at[0]], rel[q], rsem[q]).wait()

    ne_jobs = (
        (item_emb, ix_und, o_und, 0),
        (user_emb, ix_uns, o_uns, 0),
        (user_emb, ix_ind, o_ind, 1),
        (item_emb, ix_ins, o_ins, 1),
    )
    for jl, (table, ix_hbm, out_hbm, side) in enumerate(ne_jobs):
        this_v = uthis_v if side == 0 else ithis_v
        if jl % 2 == 0:
            rstart(side * RC + 0, 0)
            rstart(side * RC + 1, 1)
        pltpu.sync_copy(ix_hbm.at[w], idx_v.at[pl.ds(0, nch_ne)])
        base = w * (nch_ne * CH)

        def gstart(p, b):
            return (pltpu.async_copy(table.at[idx_v.at[2 * p]], rows[2 * b], gs[2 * b]),
                    pltpu.async_copy(table.at[idx_v.at[2 * p + 1]], rows[2 * b + 1], gs[2 * b + 1]))

        gstart(0, 0)
        gstart(1, 1)

        def body(p2, _):
            for g in (0, 1):
                p = 2 * p2 + g
                x0, x1 = 2 * g, 2 * g + 1
                pltpu.make_async_copy(table.at[idx_v.at[0]], rows[x0], gs[x0]).wait()
                pltpu.make_async_copy(table.at[idx_v.at[0]], rows[x1], gs[x1]).wait()
                w0 = pltpu.async_copy(rows[x0], out_hbm.at[pl.ds(base + 2 * p * CH, CH)], ws[x0])
                w1 = pltpu.async_copy(rows[x1], out_hbm.at[pl.ds(base + (2 * p + 1) * CH, CH)], ws[x1])
                w0.wait()
                w1.wait()
                pn = jnp.where(p + 2 >= npair, p + 2 - npair, p + 2)
                gstart(pn, g)
                tstep = (jl % 2) * npair + p2 * 2 + g
                for q in (0, 1):
                    rc = 2 * tstep + q
                    rwait(q)
                    _dot_scores(this_v, rel[q], scores_v, tpose_v, rc * nbr, nbr)
                    rcn = jnp.where(rc + 2 >= RC, rc + 2 - RC, rc + 2)
                    rstart(side * RC + rcn, q)
            return ()
        lax.fori_loop(0, npair // 2, body, (), unroll=False)
        for x in range(4):
            pltpu.make_async_copy(table.at[idx_v.at[0]], rows[x], gs[x]).wait()
        if jl % 2 == 1:
            rwait(0)
            rwait(1)
            osd = o_usd if side == 0 else o_isd
            pltpu.sync_copy(scores_v, osd.at[pl.ds(w * m * K, m * K)])


def _sc_gather_all(user_emb, item_emb, review_emb,
                   users_ind, items_ind, u_ne_items, u_ne_users, u_review_ids,
                   i_ne_users, i_ne_items, i_review_ids):
    n = users_ind.shape[0]
    m = n // NW
    nch_ne = (n * K) // (NW * CH)
    nch_rel = (n * K) // (NW * CR)
    mesh = plsc.VectorSubcoreMesh(core_axis_name="c", subcore_axis_name="s",
                                  num_cores=NC, num_subcores=NS)
    small = jax.ShapeDtypeStruct((n, H), jnp.float32)
    big = jax.ShapeDtypeStruct((n * K, H), jnp.float32)
    sd = jax.ShapeDtypeStruct((n * K,), jnp.float32)
    fn = pl.kernel(
        functools.partial(_sc_gather_body, m, nch_ne, nch_rel),
        out_type=(small, small, big, big, big, big, sd, sd),
        mesh=mesh,
        scratch_types=(
            [pltpu.VMEM((nch_ne, CH), jnp.int32),
             pltpu.VMEM((2 * nch_rel, CR), jnp.int32),
             pltpu.VMEM((1, m), jnp.int32),
             pltpu.VMEM((m, H), jnp.float32),
             pltpu.VMEM((m, H), jnp.float32),
             pltpu.VMEM((m * K,), jnp.float32),
             pltpu.VMEM((L * L,), jnp.float32)]
            + [pltpu.VMEM((CH, H), jnp.float32)] * 4
            + [pltpu.VMEM((CR, H), jnp.float32)] * 2
            + [pltpu.SemaphoreType.DMA] * 10
        ),
    )
    return fn(user_emb, item_emb, review_emb,
              users_ind.reshape(NW, 1, m), items_ind.reshape(NW, 1, m),
              u_ne_items.reshape(NW, nch_ne, CH),
              u_ne_users.reshape(NW, nch_ne, CH),
              u_review_ids.reshape(NW, nch_rel, CR),
              i_ne_users.reshape(NW, nch_ne, CH),
              i_ne_items.reshape(NW, nch_ne, CH),
              i_review_ids.reshape(NW, nch_rel, CR))


def _ln(x, g, b):
    m = jnp.mean(x, axis=-1, keepdims=True)
    v = jnp.mean((x - m) ** 2, axis=-1, keepdims=True)
    return (x - m) * jax.lax.rsqrt(v + 1e-5) * g + b


def _side(this, nd, ns, sd_raw):
    inv = 1.0 / math.sqrt(H)
    sd = sd_raw * inv
    ss = jnp.sum((this * this)[:, None, :] * ns, axis=-1) * inv
    m = jnp.maximum(jnp.max(sd, axis=-1, keepdims=True),
                    jnp.max(ss, axis=-1, keepdims=True))
    ed = jnp.exp(sd - m)
    es = jnp.exp(ss - m)
    z = jnp.sum(ed, axis=-1, keepdims=True) + jnp.sum(es, axis=-1, keepdims=True)
    pref = (jnp.sum(ed[:, :, None] * nd, axis=1)
            + jnp.sum(es[:, :, None] * ns, axis=1)) / z
    return jnp.concatenate([this, pref], axis=-1)


def _transform(x, W1, b1, W2, b2, g1, be1, g2, be2):
    h1 = jnp.maximum(jnp.dot(x, W1, preferred_element_type=jnp.float32) + b1, 0.0)
    h1 = _ln(h1, g1, be1)
    h2 = jnp.maximum(jnp.dot(h1, W2, preferred_element_type=jnp.float32) + b2, 0.0)
    return _ln(h2, g2, be2)


def _gat_block(u_this_ref, i_this_ref, u_nd_ref, u_ns_ref, i_nd_ref, i_ns_ref,
               u_sd_ref, i_sd_ref,
               W1_ref, b1_ref, W2_ref, b2_ref, g1_ref, be1_ref, g2_ref, be2_ref,
               up_ref, ip_ref, rp_ref):
    u_vec = _side(u_this_ref[...], u_nd_ref[...], u_ns_ref[...], u_sd_ref[...])
    i_vec = _side(i_this_ref[...], i_nd_ref[...], i_ns_ref[...], i_sd_ref[...])
    args = (W1_ref[...], b1_ref[...], W2_ref[...], b2_ref[...],
            g1_ref[...], be1_ref[...], g2_ref[...], be2_ref[...])
    up = _transform(u_vec, *args)
    ip = _transform(i_vec, *args)
    up_ref[...] = up
    ip_ref[...] = ip
    rp_ref[...] = up * ip


def _gat_tc(u_this, i_this, u_nd, u_ns, i_nd, i_ns, u_sd, i_sd,
            W1, b1, W2, b2, g1, be1, g2, be2):
    n = u_this.shape[0]
    nblk = n // BB
    bspec2 = pl.BlockSpec((BB, H), lambda i: (i, 0))
    bspec3 = pl.BlockSpec((BB, K, H), lambda i: (i, 0, 0))
    bspecs = pl.BlockSpec((BB, K), lambda i: (i, 0))
    wfull = lambda s: pl.BlockSpec(s, lambda i: tuple(0 for _ in s))
    out_shapes = [jax.ShapeDtypeStruct((n, H), jnp.float32)] * 3
    return pl.pallas_call(
        _gat_block,
        grid=(nblk,),
        in_specs=[bspec2, bspec2, bspec3, bspec3, bspec3, bspec3,
                  bspecs, bspecs,
                  wfull((2 * H, H)), wfull((1, H)), wfull((H, H)), wfull((1, H)),
                  wfull((1, H)), wfull((1, H)), wfull((1, H)), wfull((1, H))],
        out_specs=[bspec2, bspec2, bspec2],
        out_shape=out_shapes,
    )(u_this, i_this, u_nd, u_ns, i_nd, i_ns, u_sd, i_sd,
      W1, b1.reshape(1, H), W2, b2.reshape(1, H),
      g1.reshape(1, H), be1.reshape(1, H), g2.reshape(1, H), be2.reshape(1, H))


def kernel(users_ind, items_ind, u_ne_items, u_ne_users, i_ne_users, i_ne_items,
           u_review_ids, i_review_ids, user_emb, item_emb, review_emb,
           W1, b1, W2, b2, g1, be1, g2, be2):
    bs = B // NSPLIT
    gathered = []
    for s in range(NSPLIT):
        sl = slice(s * bs, (s + 1) * bs)
        gathered.append(_sc_gather_all(
            user_emb, item_emb, review_emb,
            users_ind[sl], items_ind[sl], u_ne_items[sl], u_ne_users[sl],
            u_review_ids[sl], i_ne_users[sl], i_ne_items[sl], i_review_ids[sl]))
    outs = []
    for s in range(NSPLIT):
        (u_this, i_this, u_nd, u_ns, i_nd, i_ns, u_sd, i_sd) = gathered[s]
        r3 = lambda a: a.reshape(bs, K, H)
        outs.append(_gat_tc(u_this, i_this, r3(u_nd), r3(u_ns),
                            r3(i_nd), r3(i_ns),
                            u_sd.reshape(bs, K), i_sd.reshape(bs, K),
                            W1, b1, W2, b2, g1, be1, g2, be2))
    return tuple(jnp.concatenate([o[j] for o in outs], axis=0) for j in range(3))


# speedup vs baseline: 1.2944x; 1.0674x over previous
import functools
import math

import jax
import jax.numpy as jnp
from jax import lax
from jax.experimental import pallas as pl
from jax.experimental.pallas import tpu as pltpu
from jax.experimental.pallas import tpu_sc as plsc

B = 4096
K = 32
H = 128
BB = 256

NC, NS = 2, 16
NW = NC * NS
CH = 128
CR = 64
L = 16
NSPLIT = 2


def _dot_scores(this_v, rows_v, scores_v, tpose_v, b0, nb):
    iota = lax.iota(jnp.int32, L)
    dn = lax.GatherDimensionNumbers(offset_dims=(), collapsed_slice_dims=(0,),
                                    start_index_map=(0,))

    def _shuf(x, sv):
        return lax.gather(x, sv[:, None], dn, slice_sizes=(1,),
                          mode=lax.GatherScatterMode.PROMISE_IN_BOUNDS)

    levels = [((iota & sh) == 0, iota ^ sh) for sh in (1, 2, 4, 8)]
    nkg = K // L

    def kgroup(u, _):
        bl = u >> 1 if nkg == 2 else u // nkg
        kg = u & 1 if nkg == 2 else u % nkg
        b = b0 + bl
        t = [this_v[b, pl.ds(L * c, L)] for c in range(H // L)]
        r0 = bl * K + kg * L
        vs = []
        for j in range(L):
            acc = t[0] * rows_v[r0 + j, pl.ds(0, L)]
            for c in range(1, H // L):
                acc += t[c] * rows_v[r0 + j, pl.ds(L * c, L)]
            vs.append(acc)
        for mlane, sv in levels:
            vs = [jnp.where(mlane, vs[i], vs[i + 1])
                  + _shuf(jnp.where(mlane, vs[i + 1], vs[i]), sv)
                  for i in range(0, len(vs), 2)]
        scores_v[pl.ds(b * K + kg * L, L)] = vs[0]
        return ()

    lax.fori_loop(0, nb * nkg, kgroup, (), unroll=False)


def _sc_gather_body(m, nch_ne, nch_rel, user_emb, item_emb, review_emb,
                    ix_uthis, ix_ithis, ix_und, ix_uns, ix_url,
                    ix_ind, ix_ins, ix_irl,
                    o_uthis, o_ithis, o_und, o_uns, o_ind, o_ins,
                    o_usd, o_isd, o_uss, o_iss,
                    idx_v, idxr_v, idxs_v, uthis_v, ithis_v, usq_v, isq_v,
                    scores_v, scores2_v, tpose_v,
                    rows0, rows1, rows2, rows3, rel0, rel1,
                    gs0, gs1, gs2, gs3, ws0, ws1, ws2, ws3, rs0, rs1):
    w = lax.axis_index("s") * NC + lax.axis_index("c")
    rows = (rows0, rows1, rows2, rows3)
    gs = (gs0, gs1, gs2, gs3)
    ws = (ws0, ws1, ws2, ws3)

    for table, ixs, out, this_v, sq_v, sem in (
            (user_emb, ix_uthis, o_uthis, uthis_v, usq_v, gs0),
            (item_emb, ix_ithis, o_ithis, ithis_v, isq_v, gs1)):
        pltpu.sync_copy(ixs.at[w], idxs_v)
        pltpu.async_copy(table.at[idxs_v.at[0]], this_v, sem).wait()
        pltpu.sync_copy(this_v, out.at[pl.ds(w * m, m)])

        def sqrow(i, _):
            for c in range(H // L):
                x = this_v[i, pl.ds(L * c, L)]
                sq_v[i, pl.ds(L * c, L)] = x * x
            return ()
        lax.fori_loop(0, m, sqrow, (), unroll=False)

    pltpu.sync_copy(ix_url.at[w], idxr_v.at[pl.ds(0, nch_rel)])
    pltpu.sync_copy(ix_irl.at[w], idxr_v.at[pl.ds(nch_rel, nch_rel)])
    nbr = CR // K
    npair = nch_ne // 2
    RC = nch_rel
    rel = (rel0, rel1)
    rsem = (rs0, rs1)

    def rstart(row, q):
        return pltpu.async_copy(review_emb.at[idxr_v.at[row]], rel[q], rsem[q])

    def rwait(q):
        pltpu.make_async_copy(review_emb.at[idxr_v.at[0]], rel[q], rsem[q]).wait()

    ne_jobs = (
        (item_emb, ix_und, o_und, 0, None),
        (user_emb, ix_uns, o_uns, 0, usq_v),
        (user_emb, ix_ind, o_ind, 1, None),
        (item_emb, ix_ins, o_ins, 1, isq_v),
    )
    for jl, (table, ix_hbm, out_hbm, side, sq_v) in enumerate(ne_jobs):
        this_v = uthis_v if side == 0 else ithis_v
        if jl % 2 == 0:
            rstart(side * RC + 0, 0)
            rstart(side * RC + 1, 1)
        pltpu.sync_copy(ix_hbm.at[w], idx_v.at[pl.ds(0, nch_ne)])
        base = w * (nch_ne * CH)

        def gstart(p, b):
            return (pltpu.async_copy(table.at[idx_v.at[2 * p]], rows[2 * b], gs[2 * b]),
                    pltpu.async_copy(table.at[idx_v.at[2 * p + 1]], rows[2 * b + 1], gs[2 * b + 1]))

        gstart(0, 0)
        gstart(1, 1)

        def body(p2, _):
            for g in (0, 1):
                p = 2 * p2 + g
                x0, x1 = 2 * g, 2 * g + 1
                pltpu.make_async_copy(table.at[idx_v.at[0]], rows[x0], gs[x0]).wait()
                pltpu.make_async_copy(table.at[idx_v.at[0]], rows[x1], gs[x1]).wait()
                w0 = pltpu.async_copy(rows[x0], out_hbm.at[pl.ds(base + 2 * p * CH, CH)], ws[x0])
                w1 = pltpu.async_copy(rows[x1], out_hbm.at[pl.ds(base + (2 * p + 1) * CH, CH)], ws[x1])
                if sq_v is not None:
                    nbc = CH // K
                    _dot_scores(sq_v, rows[x0], scores2_v, tpose_v, 2 * p * nbc, nbc)
                    _dot_scores(sq_v, rows[x1], scores2_v, tpose_v, (2 * p + 1) * nbc, nbc)
                w0.wait()
                w1.wait()
                pn = jnp.where(p + 2 >= npair, p + 2 - npair, p + 2)
                gstart(pn, g)
                tstep = (jl % 2) * npair + p2 * 2 + g
                for q in (0, 1):
                    rc = 2 * tstep + q
                    rwait(q)
                    _dot_scores(this_v, rel[q], scores_v, tpose_v, rc * nbr, nbr)
                    rcn = jnp.where(rc + 2 >= RC, rc + 2 - RC, rc + 2)
                    rstart(side * RC + rcn, q)
            return ()
        lax.fori_loop(0, npair // 2, body, (), unroll=False)
        for x in range(4):
            pltpu.make_async_copy(table.at[idx_v.at[0]], rows[x], gs[x]).wait()
        if jl % 2 == 1:
            rwait(0)
            rwait(1)
            osd = o_usd if side == 0 else o_isd
            oss = o_uss if side == 0 else o_iss
            pltpu.sync_copy(scores_v, osd.at[pl.ds(w * m * K, m * K)])
            pltpu.sync_copy(scores2_v, oss.at[pl.ds(w * m * K, m * K)])


def _sc_gather_all(user_emb, item_emb, review_emb,
                   users_ind, items_ind, u_ne_items, u_ne_users, u_review_ids,
                   i_ne_users, i_ne_items, i_review_ids):
    n = users_ind.shape[0]
    m = n // NW
    nch_ne = (n * K) // (NW * CH)
    nch_rel = (n * K) // (NW * CR)
    mesh = plsc.VectorSubcoreMesh(core_axis_name="c", subcore_axis_name="s",
                                  num_cores=NC, num_subcores=NS)
    small = jax.ShapeDtypeStruct((n, H), jnp.float32)
    big = jax.ShapeDtypeStruct((n * K, H), jnp.float32)
    sd = jax.ShapeDtypeStruct((n * K,), jnp.float32)
    fn = pl.kernel(
        functools.partial(_sc_gather_body, m, nch_ne, nch_rel),
        out_type=(small, small, big, big, big, big, sd, sd, sd, sd),
        mesh=mesh,
        scratch_types=(
            [pltpu.VMEM((nch_ne, CH), jnp.int32),
             pltpu.VMEM((2 * nch_rel, CR), jnp.int32),
             pltpu.VMEM((1, m), jnp.int32),
             pltpu.VMEM((m, H), jnp.float32),
             pltpu.VMEM((m, H), jnp.float32),
             pltpu.VMEM((m, H), jnp.float32),
             pltpu.VMEM((m, H), jnp.float32),
             pltpu.VMEM((m * K,), jnp.float32),
             pltpu.VMEM((m * K,), jnp.float32),
             pltpu.VMEM((L * L,), jnp.float32)]
            + [pltpu.VMEM((CH, H), jnp.float32)] * 4
            + [pltpu.VMEM((CR, H), jnp.float32)] * 2
            + [pltpu.SemaphoreType.DMA] * 10
        ),
    )
    return fn(user_emb, item_emb, review_emb,
              users_ind.reshape(NW, 1, m), items_ind.reshape(NW, 1, m),
              u_ne_items.reshape(NW, nch_ne, CH),
              u_ne_users.reshape(NW, nch_ne, CH),
              u_review_ids.reshape(NW, nch_rel, CR),
              i_ne_users.reshape(NW, nch_ne, CH),
              i_ne_items.reshape(NW, nch_ne, CH),
              i_review_ids.reshape(NW, nch_rel, CR))


def _ln(x, g, b):
    m = jnp.mean(x, axis=-1, keepdims=True)
    v = jnp.mean((x - m) ** 2, axis=-1, keepdims=True)
    return (x - m) * jax.lax.rsqrt(v + 1e-5) * g + b


def _side(this, nd, ns, sd_raw, ss_raw):
    inv = 1.0 / math.sqrt(H)
    sd = sd_raw * inv
    ss = ss_raw * inv
    m = jnp.maximum(jnp.max(sd, axis=-1, keepdims=True),
                    jnp.max(ss, axis=-1, keepdims=True))
    ed = jnp.exp(sd - m)
    es = jnp.exp(ss - m)
    z = jnp.sum(ed, axis=-1, keepdims=True) + jnp.sum(es, axis=-1, keepdims=True)
    pref = (jnp.sum(ed[:, :, None] * nd, axis=1)
            + jnp.sum(es[:, :, None] * ns, axis=1)) / z
    return jnp.concatenate([this, pref], axis=-1)


def _transform(x, W1, b1, W2, b2, g1, be1, g2, be2):
    h1 = jnp.maximum(jnp.dot(x, W1, preferred_element_type=jnp.float32) + b1, 0.0)
    h1 = _ln(h1, g1, be1)
    h2 = jnp.maximum(jnp.dot(h1, W2, preferred_element_type=jnp.float32) + b2, 0.0)
    return _ln(h2, g2, be2)


def _gat_block(u_this_ref, i_this_ref, u_nd_ref, u_ns_ref, i_nd_ref, i_ns_ref,
               u_sd_ref, i_sd_ref, u_ss_ref, i_ss_ref,
               W1_ref, b1_ref, W2_ref, b2_ref, g1_ref, be1_ref, g2_ref, be2_ref,
               up_ref, ip_ref, rp_ref):
    u_vec = _side(u_this_ref[...], u_nd_ref[...], u_ns_ref[...],
                  u_sd_ref[...], u_ss_ref[...])
    i_vec = _side(i_this_ref[...], i_nd_ref[...], i_ns_ref[...],
                  i_sd_ref[...], i_ss_ref[...])
    args = (W1_ref[...], b1_ref[...], W2_ref[...], b2_ref[...],
            g1_ref[...], be1_ref[...], g2_ref[...], be2_ref[...])
    up = _transform(u_vec, *args)
    ip = _transform(i_vec, *args)
    up_ref[...] = up
    ip_ref[...] = ip
    rp_ref[...] = up * ip


def _gat_tc(u_this, i_this, u_nd, u_ns, i_nd, i_ns, u_sd, i_sd, u_ss, i_ss,
            W1, b1, W2, b2, g1, be1, g2, be2):
    n = u_this.shape[0]
    nblk = n // BB
    bspec2 = pl.BlockSpec((BB, H), lambda i: (i, 0))
    bspec3 = pl.BlockSpec((BB, K, H), lambda i: (i, 0, 0))
    bspecs = pl.BlockSpec((BB, K), lambda i: (i, 0))
    wfull = lambda s: pl.BlockSpec(s, lambda i: tuple(0 for _ in s))
    out_shapes = [jax.ShapeDtypeStruct((n, H), jnp.float32)] * 3
    return pl.pallas_call(
        _gat_block,
        grid=(nblk,),
        in_specs=[bspec2, bspec2, bspec3, bspec3, bspec3, bspec3,
                  bspecs, bspecs, bspecs, bspecs,
                  wfull((2 * H, H)), wfull((1, H)), wfull((H, H)), wfull((1, H)),
                  wfull((1, H)), wfull((1, H)), wfull((1, H)), wfull((1, H))],
        out_specs=[bspec2, bspec2, bspec2],
        out_shape=out_shapes,
    )(u_this, i_this, u_nd, u_ns, i_nd, i_ns, u_sd, i_sd, u_ss, i_ss,
      W1, b1.reshape(1, H), W2, b2.reshape(1, H),
      g1.reshape(1, H), be1.reshape(1, H), g2.reshape(1, H), be2.reshape(1, H))


def kernel(users_ind, items_ind, u_ne_items, u_ne_users, i_ne_users, i_ne_items,
           u_review_ids, i_review_ids, user_emb, item_emb, review_emb,
           W1, b1, W2, b2, g1, be1, g2, be2):
    bs = B // NSPLIT
    gathered = []
    for s in range(NSPLIT):
        sl = slice(s * bs, (s + 1) * bs)
        gathered.append(_sc_gather_all(
            user_emb, item_emb, review_emb,
            users_ind[sl], items_ind[sl], u_ne_items[sl], u_ne_users[sl],
            u_review_ids[sl], i_ne_users[sl], i_ne_items[sl], i_review_ids[sl]))
    outs = []
    for s in range(NSPLIT):
        (u_this, i_this, u_nd, u_ns, i_nd, i_ns,
         u_sd, i_sd, u_ss, i_ss) = gathered[s]
        r3 = lambda a: a.reshape(bs, K, H)
        outs.append(_gat_tc(u_this, i_this, r3(u_nd), r3(u_ns),
                            r3(i_nd), r3(i_ns),
                            u_sd.reshape(bs, K), i_sd.reshape(bs, K),
                            u_ss.reshape(bs, K), i_ss.reshape(bs, K),
                            W1, b1, W2, b2, g1, be1, g2, be2))
    return tuple(jnp.concatenate([o[j] for o in outs], axis=0) for j in range(3))
